# baseline (speedup 1.0000x reference)
import jax
import jax.numpy as jnp
from jax.experimental import pallas as pl
from jax.experimental.pallas import tpu as pltpu

_KC = 256


def _support_body(a_ref, x_ref, k_ref, o_ref, acc_ref):
    n = a_ref.shape[2]
    acc_ref[...] = jnp.dot(a_ref[0, :, 0:_KC], x_ref[0, 0:_KC, :],
                           preferred_element_type=jnp.float32)
    for kc in range(1, n // _KC):
        acc_ref[...] = acc_ref[...] + jnp.dot(
            a_ref[0, :, kc * _KC:(kc + 1) * _KC],
            x_ref[0, kc * _KC:(kc + 1) * _KC, :],
            preferred_element_type=jnp.float32)
    o_ref[0] = jnp.dot(acc_ref[...], k_ref[...],
                       preferred_element_type=jnp.float32)


def _topk_body(s_row_ref, s_col_ref, kv_ref, ki_ref, rank_ref):
    n = s_row_ref.shape[2]
    k = n // 2
    ic_sz = 512
    s_row = s_row_ref[0]
    m = jnp.max(s_row)
    u_row = jnp.exp(s_row - m)
    c = jnp.sum(u_row)
    q_row = u_row / c
    iota_row = jax.lax.broadcasted_iota(jnp.int32, (1, n), 1)
    s_col = s_col_ref[0]
    q_col = jnp.exp(s_col - m) / c

    for ci in range(n // ic_sz):
        qc = q_col[ci * ic_sz:(ci + 1) * ic_sz, :]
        icol = jax.lax.broadcasted_iota(jnp.int32, (ic_sz, 1), 0) + ci * ic_sz
        gt = (q_row > qc)
        eqlow = (q_row == qc) & (iota_row < icol)
        rank = jnp.sum(gt.astype(jnp.int32) + eqlow.astype(jnp.int32),
                       axis=1, keepdims=True)
        rank_ref[ci * ic_sz:(ci + 1) * ic_sz, :] = rank

    p_row = jax.lax.broadcasted_iota(jnp.int32, (1, k), 1)
    kv_acc = jnp.zeros((1, k), jnp.float32)
    ki_acc = jnp.zeros((1, k), jnp.int32)
    for ci in range(n // ic_sz):
        qc = q_col[ci * ic_sz:(ci + 1) * ic_sz, :]
        icol = jax.lax.broadcasted_iota(jnp.int32, (ic_sz, 1), 0) + ci * ic_sz
        rank_c = rank_ref[ci * ic_sz:(ci + 1) * ic_sz, :]
        hit = (rank_c == p_row)
        kv_acc = kv_acc + jnp.sum(jnp.where(hit, qc, 0.0), axis=0, keepdims=True)
        ki_acc = ki_acc + jnp.sum(jnp.where(hit, icol, 0), axis=0, keepdims=True)
    kv_ref[0] = kv_acc
    ki_ref[0] = ki_acc


def kernel(Xs, As, attn_kernel):
    B, N, F = Xs.shape
    K = N // 2
    BLK = 512
    support = pl.pallas_call(
        _support_body,
        grid=(B, N // BLK),
        in_specs=[
            pl.BlockSpec((1, BLK, N), lambda b, i: (b, i, 0)),
            pl.BlockSpec((1, N, F), lambda b, i: (b, 0, 0)),
            pl.BlockSpec((F, 1), lambda b, i: (0, 0)),
        ],
        out_specs=pl.BlockSpec((1, BLK, 1), lambda b, i: (b, i, 0)),
        out_shape=jax.ShapeDtypeStruct((B, N, 1), jnp.float32),
        scratch_shapes=[pltpu.VMEM((BLK, F), jnp.float32)],
    )(As, Xs, attn_kernel)

    s_col = support
    s_row = jnp.reshape(support, (B, 1, N))
    keep_values, keep_indices = pl.pallas_call(
        _topk_body,
        grid=(B,),
        in_specs=[
            pl.BlockSpec((1, 1, N), lambda b: (b, 0, 0)),
            pl.BlockSpec((1, N, 1), lambda b: (b, 0, 0)),
        ],
        out_specs=[
            pl.BlockSpec((1, 1, K), lambda b: (b, 0, 0)),
            pl.BlockSpec((1, 1, K), lambda b: (b, 0, 0)),
        ],
        out_shape=[
            jax.ShapeDtypeStruct((B, 1, K), jnp.float32),
            jax.ShapeDtypeStruct((B, 1, K), jnp.int32),
        ],
        scratch_shapes=[pltpu.VMEM((N, 1), jnp.int32)],
    )(s_row, s_col)
    keep_values = jnp.reshape(keep_values, (B, K))
    keep_indices = jnp.reshape(keep_indices, (B, K))

    Xs_out = jnp.take_along_axis(Xs, keep_indices[:, :, None], axis=1)
    A_rows = jnp.take_along_axis(As, keep_indices[:, :, None], axis=1)
    As_out = jnp.take_along_axis(A_rows, keep_indices[:, None, :], axis=2)
    return (Xs_out, As_out, keep_values)
